# baseline (speedup 1.0000x reference)
import functools

import jax
import jax.numpy as jnp
from jax import lax
from jax.experimental import pallas as pl
from jax.experimental.pallas import tpu as pltpu
from jax.experimental.pallas import tpu_sc as plsc

N = 10000
D = 128
E = 320000

EPS = 1e-7
MAXN = 1.0 - 1e-5

NC, NS = 2, 16
NW = NC * NS
K = 64
SUPER = 16
NBUF = 4
CH0 = 320
CHUNKS_TOTAL = NS * CH0
E_PAD = K * CHUNKS_TOTAL
N_PAD = 10240
ROWS_PER_TILE = N_PAD // NS


def _tangent(x):
    n = jnp.sqrt(jnp.sum(x * x, axis=-1, keepdims=True))
    n = jnp.maximum(n, EPS)
    xp = jnp.where(n > MAXN, x / n * MAXN, x)
    n2 = jnp.maximum(jnp.sqrt(jnp.sum(xp * xp, axis=-1, keepdims=True)), EPS)
    arg = jnp.minimum(n2, MAXN)
    atanh = 0.5 * (jnp.log1p(arg) - jnp.log1p(-arg))
    return xp / n2 * atanh


def _combine(p, d, h):
    deg = lax.dot_general(d, jnp.ones((NS, 1), jnp.float32),
                          (((0,), (0,)), ((), ())),
                          preferred_element_type=jnp.float32)
    deg = jnp.maximum(deg, 1.0)
    y = jax.nn.relu(p / deg + h)
    n = jnp.maximum(jnp.sqrt(jnp.sum(y * y, axis=-1, keepdims=True)), EPS)
    e = jnp.tanh(n) * y / n
    ne = jnp.maximum(jnp.sqrt(jnp.sum(e * e, axis=-1, keepdims=True)), EPS)
    return jnp.where(ne > MAXN, e / ne * MAXN, e)


BLK = 1024


def _pre_body(x_ref, w_ref, b_ref, o_ref):
    h = _tangent(x_ref[...])
    o_ref[...] = (
        jnp.dot(h, w_ref[...], preferred_element_type=jnp.float32) + b_ref[...]
    )


def _tc_pre(x, w, b):
    return pl.pallas_call(
        _pre_body,
        grid=(pl.cdiv(N, BLK),),
        in_specs=[
            pl.BlockSpec((BLK, D), lambda i: (i, 0)),
            pl.BlockSpec((D, D), lambda i: (0, 0)),
            pl.BlockSpec((1, D), lambda i: (0, 0)),
        ],
        out_specs=pl.BlockSpec((BLK, D), lambda i: (i, 0)),
        out_shape=jax.ShapeDtypeStruct((N, D), jnp.float32),
    )(x, w, b.reshape(1, D))


def _combine_pre_body(p_ref, d_ref, h_ref, w_ref, b_ref, o_ref):
    y = _combine(p_ref[...], d_ref[...], h_ref[...])
    h2 = _tangent(y)
    o_ref[...] = (
        jnp.dot(h2, w_ref[...], preferred_element_type=jnp.float32) + b_ref[...]
    )


def _tc_combine_pre(parts, degs, h, w, b):
    return pl.pallas_call(
        _combine_pre_body,
        grid=(pl.cdiv(N, BLK),),
        in_specs=[
            pl.BlockSpec((BLK, D), lambda i: (i, 0)),
            pl.BlockSpec((NS, BLK), lambda i: (0, i)),
            pl.BlockSpec((BLK, D), lambda i: (i, 0)),
            pl.BlockSpec((D, D), lambda i: (0, 0)),
            pl.BlockSpec((1, D), lambda i: (0, 0)),
        ],
        out_specs=pl.BlockSpec((BLK, D), lambda i: (i, 0)),
        out_shape=jax.ShapeDtypeStruct((N, D), jnp.float32),
    )(parts, degs, h, w, b.reshape(1, D))


def _combine_body(p_ref, d_ref, h_ref, o_ref):
    o_ref[...] = _combine(p_ref[...], d_ref[...], h_ref[...])


def _tc_combine(parts, degs, h):
    return pl.pallas_call(
        _combine_body,
        grid=(pl.cdiv(N, BLK),),
        in_specs=[
            pl.BlockSpec((BLK, D), lambda i: (i, 0)),
            pl.BlockSpec((NS, BLK), lambda i: (0, i)),
            pl.BlockSpec((BLK, D), lambda i: (i, 0)),
        ],
        out_specs=pl.BlockSpec((BLK, D), lambda i: (i, 0)),
        out_shape=jax.ShapeDtypeStruct((N, D), jnp.float32),
    )(parts, degs, h)


def _sc_body(h_hbm, src_hbm, dst_hbm, zrow_hbm, zdeg_hbm,
             acc_out, deg_out,
             src_v, dst_v, rows, deg_v, acc_sh, gsems, ssems):
    cid = lax.axis_index("c")
    sid = lax.axis_index("s")

    ones16 = jnp.full((16,), 1.0, jnp.float32)

    def gstart(j):
        pltpu.async_copy(h_hbm.at[src_v.at[j]], rows[j % NBUF], gsems[j % NBUF])

    def gwait(j):
        pltpu.make_async_copy(h_hbm.at[src_v.at[j]], rows[j % NBUF],
                              gsems[j % NBUF]).wait()

    def sstart(j):
        pltpu.async_copy(rows[j % NBUF], acc_sh.at[dst_v.at[j]],
                         ssems[j % NBUF], add=True)

    def swait(j):
        pltpu.make_async_copy(rows[j % NBUF], acc_sh.at[dst_v.at[j]],
                              ssems[j % NBUF]).wait()

    def hist(j):
        for t in range(K // 16):
            plsc.addupdate_scatter(deg_v, [dst_v[j, pl.ds(t * 16, 16)]], ones16)

    def make_super_group(base_ch):
        def super_group(s, carry):
            base = base_ch + s * SUPER
            pltpu.sync_copy(src_hbm.at[pl.ds(base, SUPER)], src_v)
            pltpu.sync_copy(dst_hbm.at[pl.ds(base, SUPER)], dst_v)
            for j in range(NBUF - 1):
                gstart(j)
            for j in range(SUPER):
                gwait(j)
                if j > 0:
                    swait(j - 1)
                if j + NBUF - 1 < SUPER:
                    gstart(j + NBUF - 1)
                sstart(j)
                hist(j)
            swait(SUPER - 1)
            return carry
        return super_group

    @pl.when(cid == 0)
    def _():
        pltpu.sync_copy(zrow_hbm,
                        acc_sh.at[pl.ds(sid * ROWS_PER_TILE, ROWS_PER_TILE)])
        pltpu.sync_copy(zdeg_hbm, deg_v)
        plsc.subcore_barrier()
        lax.fori_loop(0, CH0 // SUPER, make_super_group(sid * CH0), 0)
        plsc.subcore_barrier()
        pltpu.sync_copy(acc_sh.at[pl.ds(sid * ROWS_PER_TILE, ROWS_PER_TILE)],
                        acc_out.at[pl.ds(sid * ROWS_PER_TILE, ROWS_PER_TILE)])
        pltpu.sync_copy(deg_v, deg_out.at[sid])


@functools.cache
def _sc_aggregate_kernel():
    return pl.kernel(
        _sc_body,
        out_type=[
            jax.ShapeDtypeStruct((N_PAD, D), jnp.float32),
            jax.ShapeDtypeStruct((NS, N_PAD), jnp.float32),
        ],
        mesh=plsc.VectorSubcoreMesh(core_axis_name="c", subcore_axis_name="s",
                                    num_cores=NC, num_subcores=NS),
        compiler_params=pltpu.CompilerParams(needs_layout_passes=False),
        scratch_types=[
            pltpu.VMEM((SUPER, K), jnp.int32),
            pltpu.VMEM((SUPER, K), jnp.int32),
            [pltpu.VMEM((K, D), jnp.float32)] * NBUF,
            pltpu.VMEM((N_PAD,), jnp.float32),
            pltpu.VMEM_SHARED((N_PAD, D), jnp.float32),
            [pltpu.SemaphoreType.DMA] * NBUF,
            [pltpu.SemaphoreType.DMA] * NBUF,
        ],
    )


def _pad_edges(src, dst):
    pad = E_PAD - E
    pad_src = jnp.zeros((pad,), jnp.int32)
    pad_dst = N + (jnp.arange(pad, dtype=jnp.int32) % (N_PAD - N))
    return (jnp.concatenate([src.astype(jnp.int32), pad_src]).reshape(-1, K),
            jnp.concatenate([dst.astype(jnp.int32), pad_dst]).reshape(-1, K))


def kernel(x, adj, W1, b1, W2, b2):
    zrow = jnp.zeros((ROWS_PER_TILE, D), jnp.float32)
    zdeg = jnp.zeros((N_PAD,), jnp.float32)

    src1, dst1 = _pad_edges(adj[0, 0], adj[0, 1])
    src2, dst2 = _pad_edges(adj[1, 0], adj[1, 1])

    sc_aggregate = _sc_aggregate_kernel()
    h1 = _tc_pre(x, W1, b1)
    acc1, deg1 = sc_aggregate(h1, src1, dst1, zrow, zdeg)
    h2 = _tc_combine_pre(acc1, deg1, h1, W2, b2)
    acc2, deg2 = sc_aggregate(h2, src2, dst2, zrow, zdeg)
    return _tc_combine(acc2, deg2, h2)


# speedup vs baseline: 1.3168x; 1.3168x over previous
import functools

import jax
import jax.numpy as jnp
from jax import lax
from jax.experimental import pallas as pl
from jax.experimental.pallas import tpu as pltpu
from jax.experimental.pallas import tpu_sc as plsc

N = 10000
D = 128
E = 320000

EPS = 1e-7
MAXN = 1.0 - 1e-5

NC, NS = 2, 16
NW = NC * NS
K = 64
SUPER = 16
NBUF = 4
CH0 = 160
CH1 = 160
CHUNKS_TOTAL = NS * (CH0 + CH1)
E_PAD = K * CHUNKS_TOTAL
N_PAD = 10240
ROWS_PER_TILE = N_PAD // NS


def _tangent(x):
    n = jnp.sqrt(jnp.sum(x * x, axis=-1, keepdims=True))
    n = jnp.maximum(n, EPS)
    xp = jnp.where(n > MAXN, x / n * MAXN, x)
    n2 = jnp.maximum(jnp.sqrt(jnp.sum(xp * xp, axis=-1, keepdims=True)), EPS)
    arg = jnp.minimum(n2, MAXN)
    atanh = 0.5 * (jnp.log1p(arg) - jnp.log1p(-arg))
    return xp / n2 * atanh


def _combine(p0, p1, d, h):
    deg = lax.dot_general(d, jnp.ones((NW, 1), jnp.float32),
                          (((0,), (0,)), ((), ())),
                          preferred_element_type=jnp.float32)
    deg = jnp.maximum(deg, 1.0)
    y = jax.nn.relu((p0 + p1) / deg + h)
    n = jnp.maximum(jnp.sqrt(jnp.sum(y * y, axis=-1, keepdims=True)), EPS)
    e = jnp.tanh(n) * y / n
    ne = jnp.maximum(jnp.sqrt(jnp.sum(e * e, axis=-1, keepdims=True)), EPS)
    return jnp.where(ne > MAXN, e / ne * MAXN, e)


BLK = 1024


def _pre_body(x_ref, w_ref, b_ref, o_ref):
    h = _tangent(x_ref[...])
    o_ref[...] = (
        jnp.dot(h, w_ref[...], preferred_element_type=jnp.float32) + b_ref[...]
    )


def _tc_pre(x, w, b):
    return pl.pallas_call(
        _pre_body,
        grid=(pl.cdiv(N, BLK),),
        in_specs=[
            pl.BlockSpec((BLK, D), lambda i: (i, 0)),
            pl.BlockSpec((D, D), lambda i: (0, 0)),
            pl.BlockSpec((1, D), lambda i: (0, 0)),
        ],
        out_specs=pl.BlockSpec((BLK, D), lambda i: (i, 0)),
        out_shape=jax.ShapeDtypeStruct((N, D), jnp.float32),
    )(x, w, b.reshape(1, D))


def _combine_pre_body(p_ref0, p_ref1, d_ref, h_ref, w_ref, b_ref, o_ref):
    y = _combine(p_ref0[0], p_ref1[0], d_ref[...], h_ref[...])
    h2 = _tangent(y)
    o_ref[...] = (
        jnp.dot(h2, w_ref[...], preferred_element_type=jnp.float32) + b_ref[...]
    )


def _tc_combine_pre(parts, degs, h, w, b):
    return pl.pallas_call(
        _combine_pre_body,
        grid=(pl.cdiv(N, BLK),),
        in_specs=[
            pl.BlockSpec((1, BLK, D), lambda i: (0, i, 0)),
            pl.BlockSpec((1, BLK, D), lambda i: (1, i, 0)),
            pl.BlockSpec((NW, BLK), lambda i: (0, i)),
            pl.BlockSpec((BLK, D), lambda i: (i, 0)),
            pl.BlockSpec((D, D), lambda i: (0, 0)),
            pl.BlockSpec((1, D), lambda i: (0, 0)),
        ],
        out_specs=pl.BlockSpec((BLK, D), lambda i: (i, 0)),
        out_shape=jax.ShapeDtypeStruct((N, D), jnp.float32),
    )(parts, parts, degs, h, w, b.reshape(1, D))


def _combine_body(p_ref0, p_ref1, d_ref, h_ref, o_ref):
    o_ref[...] = _combine(p_ref0[0], p_ref1[0], d_ref[...], h_ref[...])


def _tc_combine(parts, degs, h):
    return pl.pallas_call(
        _combine_body,
        grid=(pl.cdiv(N, BLK),),
        in_specs=[
            pl.BlockSpec((1, BLK, D), lambda i: (0, i, 0)),
            pl.BlockSpec((1, BLK, D), lambda i: (1, i, 0)),
            pl.BlockSpec((NW, BLK), lambda i: (0, i)),
            pl.BlockSpec((BLK, D), lambda i: (i, 0)),
        ],
        out_specs=pl.BlockSpec((BLK, D), lambda i: (i, 0)),
        out_shape=jax.ShapeDtypeStruct((N, D), jnp.float32),
    )(parts, parts, degs, h)


def _sc_body(h_hbm, src_hbm, dst_hbm, zrow_hbm, zdeg_hbm,
             acc_out, deg_out,
             src_v, dst_v, rows, deg_v, acc_sh, gsems, ssems):
    cid = lax.axis_index("c")
    sid = lax.axis_index("s")
    wid = cid * NS + sid

    pltpu.sync_copy(zrow_hbm, acc_sh.at[pl.ds(sid * ROWS_PER_TILE, ROWS_PER_TILE)])
    pltpu.sync_copy(zdeg_hbm, deg_v)
    plsc.subcore_barrier()

    ones16 = jnp.full((16,), 1.0, jnp.float32)

    def gstart(j):
        pltpu.async_copy(h_hbm.at[src_v.at[j]], rows[j % NBUF], gsems[j % NBUF])

    def gwait(j):
        pltpu.make_async_copy(h_hbm.at[src_v.at[j]], rows[j % NBUF],
                              gsems[j % NBUF]).wait()

    def sstart(j):
        pltpu.async_copy(rows[j % NBUF], acc_sh.at[dst_v.at[j]],
                         ssems[j % NBUF], add=True)

    def swait(j):
        pltpu.make_async_copy(rows[j % NBUF], acc_sh.at[dst_v.at[j]],
                              ssems[j % NBUF]).wait()

    def hist(j):
        for t in range(K // 16):
            plsc.addupdate_scatter(deg_v, [dst_v[j, pl.ds(t * 16, 16)]], ones16)

    def make_super_group(base_ch):
        def super_group(s, carry):
            base = base_ch + s * SUPER
            pltpu.sync_copy(src_hbm.at[pl.ds(base, SUPER)], src_v)
            pltpu.sync_copy(dst_hbm.at[pl.ds(base, SUPER)], dst_v)
            for j in range(NBUF - 1):
                gstart(j)
            for j in range(SUPER):
                gwait(j)
                if j > 0:
                    swait(j - 1)
                if j + NBUF - 1 < SUPER:
                    gstart(j + NBUF - 1)
                sstart(j)
                hist(j)
            swait(SUPER - 1)
            return carry
        return super_group

    @pl.when(cid == 0)
    def _():
        lax.fori_loop(0, CH0 // SUPER, make_super_group(sid * CH0), 0)

    @pl.when(cid == 1)
    def _():
        lax.fori_loop(0, CH1 // SUPER, make_super_group(NS * CH0 + sid * CH1), 0)

    plsc.subcore_barrier()

    out_base = cid * N_PAD + sid * ROWS_PER_TILE
    pltpu.sync_copy(acc_sh.at[pl.ds(sid * ROWS_PER_TILE, ROWS_PER_TILE)],
                    acc_out.at[pl.ds(out_base, ROWS_PER_TILE)])
    pltpu.sync_copy(deg_v, deg_out.at[wid])


@functools.cache
def _sc_aggregate_kernel():
    return pl.kernel(
        _sc_body,
        out_type=[
            jax.ShapeDtypeStruct((NC * N_PAD, D), jnp.float32),
            jax.ShapeDtypeStruct((NW, N_PAD), jnp.float32),
        ],
        mesh=plsc.VectorSubcoreMesh(core_axis_name="c", subcore_axis_name="s",
                                    num_cores=NC, num_subcores=NS),
        compiler_params=pltpu.CompilerParams(needs_layout_passes=False),
        scratch_types=[
            pltpu.VMEM((SUPER, K), jnp.int32),
            pltpu.VMEM((SUPER, K), jnp.int32),
            [pltpu.VMEM((K, D), jnp.float32)] * NBUF,
            pltpu.VMEM((N_PAD,), jnp.float32),
            pltpu.VMEM_SHARED((N_PAD, D), jnp.float32),
            [pltpu.SemaphoreType.DMA] * NBUF,
            [pltpu.SemaphoreType.DMA] * NBUF,
        ],
    )


def _pad_edges(src, dst):
    per_w = E // NW
    pad_w = E_PAD // NW - per_w
    pad_src = jnp.zeros((NW, pad_w), jnp.int32)
    pad_dst = jnp.broadcast_to(N + jnp.arange(pad_w, dtype=jnp.int32),
                               (NW, pad_w))
    sp = jnp.concatenate([src.astype(jnp.int32).reshape(NW, per_w), pad_src], 1)
    dp = jnp.concatenate([dst.astype(jnp.int32).reshape(NW, per_w), pad_dst], 1)
    return sp.reshape(-1, K), dp.reshape(-1, K)


def kernel(x, adj, W1, b1, W2, b2):
    zrow = jnp.zeros((ROWS_PER_TILE, D), jnp.float32)
    zdeg = jnp.zeros((N_PAD,), jnp.float32)

    src1, dst1 = _pad_edges(adj[0, 0], adj[0, 1])
    src2, dst2 = _pad_edges(adj[1, 0], adj[1, 1])

    sc_aggregate = _sc_aggregate_kernel()
    h1 = _tc_pre(x, W1, b1)
    acc1, deg1 = sc_aggregate(h1, src1, dst1, zrow, zdeg)
    h2 = _tc_combine_pre(acc1.reshape(NC, N_PAD, D), deg1, h1, W2, b2)
    acc2, deg2 = sc_aggregate(h2, src2, dst2, zrow, zdeg)
    return _tc_combine(acc2.reshape(NC, N_PAD, D), deg2, h2)


# speedup vs baseline: 3.4933x; 2.6529x over previous
import functools

import jax
import jax.numpy as jnp
from jax import lax
from jax.experimental import pallas as pl
from jax.experimental.pallas import tpu as pltpu
from jax.experimental.pallas import tpu_sc as plsc

N = 10000
D = 128
E = 320000

EPS = 1e-7
MAXN = 1.0 - 1e-5

NC, NS = 2, 16
NW = NC * NS
K = 64
SUPER = 16
NBUF = 4
CH0 = 160
CH1 = 160
CHUNKS_TOTAL = NS * (CH0 + CH1)
E_PAD = K * CHUNKS_TOTAL
N_PAD = 10240
ROWS_PER_TILE = N_PAD // NS


def _tangent(x):
    n = jnp.sqrt(jnp.sum(x * x, axis=-1, keepdims=True))
    n = jnp.maximum(n, EPS)
    xp = jnp.where(n > MAXN, x / n * MAXN, x)
    n2 = jnp.maximum(jnp.sqrt(jnp.sum(xp * xp, axis=-1, keepdims=True)), EPS)
    arg = jnp.minimum(n2, MAXN)
    atanh = 0.5 * (jnp.log1p(arg) - jnp.log1p(-arg))
    return xp / n2 * atanh


def _combine(p0, p1, d, h):
    deg = lax.dot_general(d, jnp.ones((NW, 1), jnp.float32),
                          (((0,), (0,)), ((), ())),
                          preferred_element_type=jnp.float32)
    deg = jnp.maximum(deg, 1.0)
    y = jax.nn.relu((p0 + p1) / deg + h)
    n = jnp.maximum(jnp.sqrt(jnp.sum(y * y, axis=-1, keepdims=True)), EPS)
    e = jnp.tanh(n) * y / n
    ne = jnp.maximum(jnp.sqrt(jnp.sum(e * e, axis=-1, keepdims=True)), EPS)
    return jnp.where(ne > MAXN, e / ne * MAXN, e)


BLK = 1024


def _pre_body(x_ref, w_ref, b_ref, o_ref):
    h = _tangent(x_ref[...])
    o_ref[...] = (
        jnp.dot(h, w_ref[...], preferred_element_type=jnp.float32) + b_ref[...]
    )


def _tc_pre(x, w, b):
    return pl.pallas_call(
        _pre_body,
        grid=(pl.cdiv(N, BLK),),
        in_specs=[
            pl.BlockSpec((BLK, D), lambda i: (i, 0)),
            pl.BlockSpec((D, D), lambda i: (0, 0)),
            pl.BlockSpec((1, D), lambda i: (0, 0)),
        ],
        out_specs=pl.BlockSpec((BLK, D), lambda i: (i, 0)),
        out_shape=jax.ShapeDtypeStruct((N, D), jnp.float32),
    )(x, w, b.reshape(1, D))


def _combine_pre_body(p_ref0, p_ref1, d_ref, h_ref, w_ref, b_ref, o_ref):
    y = _combine(p_ref0[0], p_ref1[0], d_ref[...], h_ref[...])
    h2 = _tangent(y)
    o_ref[...] = (
        jnp.dot(h2, w_ref[...], preferred_element_type=jnp.float32) + b_ref[...]
    )


def _tc_combine_pre(parts, degs, h, w, b):
    return pl.pallas_call(
        _combine_pre_body,
        grid=(pl.cdiv(N, BLK),),
        in_specs=[
            pl.BlockSpec((1, BLK, D), lambda i: (0, i, 0)),
            pl.BlockSpec((1, BLK, D), lambda i: (1, i, 0)),
            pl.BlockSpec((NW, BLK), lambda i: (0, i)),
            pl.BlockSpec((BLK, D), lambda i: (i, 0)),
            pl.BlockSpec((D, D), lambda i: (0, 0)),
            pl.BlockSpec((1, D), lambda i: (0, 0)),
        ],
        out_specs=pl.BlockSpec((BLK, D), lambda i: (i, 0)),
        out_shape=jax.ShapeDtypeStruct((N, D), jnp.float32),
    )(parts, parts, degs, h, w, b.reshape(1, D))


def _combine_body(p_ref0, p_ref1, d_ref, h_ref, o_ref):
    o_ref[...] = _combine(p_ref0[0], p_ref1[0], d_ref[...], h_ref[...])


def _tc_combine(parts, degs, h):
    return pl.pallas_call(
        _combine_body,
        grid=(pl.cdiv(N, BLK),),
        in_specs=[
            pl.BlockSpec((1, BLK, D), lambda i: (0, i, 0)),
            pl.BlockSpec((1, BLK, D), lambda i: (1, i, 0)),
            pl.BlockSpec((NW, BLK), lambda i: (0, i)),
            pl.BlockSpec((BLK, D), lambda i: (i, 0)),
        ],
        out_specs=pl.BlockSpec((BLK, D), lambda i: (i, 0)),
        out_shape=jax.ShapeDtypeStruct((N, D), jnp.float32),
    )(parts, parts, degs, h)


def _sc_body(h_hbm, src_hbm, dst_hbm, zrow_hbm, zdeg_hbm,
             acc_out, deg_out,
             src_v, dst_v, rows, deg_v, acc_sh, gsems, ssems):
    cid = lax.axis_index("c")
    sid = lax.axis_index("s")
    wid = cid * NS + sid

    pltpu.sync_copy(zrow_hbm, acc_sh.at[pl.ds(sid * ROWS_PER_TILE, ROWS_PER_TILE)])
    pltpu.sync_copy(zdeg_hbm, deg_v)
    plsc.subcore_barrier()

    ones16 = jnp.full((16,), 1.0, jnp.float32)

    def gstart(j):
        pltpu.async_copy(h_hbm.at[src_v.at[j]], rows[j % NBUF], gsems[j % NBUF])

    def gwait(j):
        pltpu.make_async_copy(h_hbm.at[src_v.at[j]], rows[j % NBUF],
                              gsems[j % NBUF]).wait()

    def sstart(j):
        pltpu.async_copy(rows[j % NBUF], acc_sh.at[dst_v.at[j]],
                         ssems[j % NBUF], add=True)

    def swait(j):
        pltpu.make_async_copy(rows[j % NBUF], acc_sh.at[dst_v.at[j]],
                              ssems[j % NBUF]).wait()

    def hist(j):
        for t in range(K // 16):
            plsc.addupdate_scatter(deg_v, [dst_v[j, pl.ds(t * 16, 16)]], ones16)

    def make_super_group(base_ch):
        def super_group(s, carry):
            base = base_ch + s * SUPER
            pltpu.sync_copy(src_hbm.at[pl.ds(base, SUPER)], src_v)
            pltpu.sync_copy(dst_hbm.at[pl.ds(base, SUPER)], dst_v)
            for j in range(NBUF - 1):
                gstart(j)
            for j in range(SUPER):
                gwait(j)
                if j > 0:
                    swait(j - 1)
                if j + NBUF - 1 < SUPER:
                    gstart(j + NBUF - 1)
                sstart(j)
                hist(j)
            swait(SUPER - 1)
            return carry
        return super_group

    @pl.when(cid == 0)
    def _():
        lax.fori_loop(0, CH0 // SUPER, make_super_group(sid * CH0), 0)

    @pl.when(cid == 1)
    def _():
        lax.fori_loop(0, CH1 // SUPER, make_super_group(NS * CH0 + sid * CH1), 0)

    plsc.subcore_barrier()

    out_base = cid * N_PAD + sid * ROWS_PER_TILE
    pltpu.sync_copy(acc_sh.at[pl.ds(sid * ROWS_PER_TILE, ROWS_PER_TILE)],
                    acc_out.at[pl.ds(out_base, ROWS_PER_TILE)])
    pltpu.sync_copy(deg_v, deg_out.at[wid])


@functools.cache
def _sc_aggregate_kernel():
    return pl.kernel(
        _sc_body,
        out_type=[
            jax.ShapeDtypeStruct((NC * N_PAD, D), jnp.float32),
            jax.ShapeDtypeStruct((NW, N_PAD), jnp.float32),
        ],
        mesh=plsc.VectorSubcoreMesh(core_axis_name="c", subcore_axis_name="s",
                                    num_cores=NC, num_subcores=NS),
        compiler_params=pltpu.CompilerParams(needs_layout_passes=False),
        scratch_types=[
            pltpu.VMEM((SUPER, K), jnp.int32),
            pltpu.VMEM((SUPER, K), jnp.int32),
            [pltpu.VMEM((K, D), jnp.float32)] * NBUF,
            pltpu.VMEM((N_PAD,), jnp.float32),
            pltpu.VMEM_SHARED((N_PAD, D), jnp.float32),
            [pltpu.SemaphoreType.DMA] * NBUF,
            [pltpu.SemaphoreType.DMA] * NBUF,
        ],
    )


def _pad_edges(src, dst):
    per_w = E // NW
    pad_w = E_PAD // NW - per_w
    i = jnp.arange(pad_w, dtype=jnp.int32)
    w = jnp.arange(NW, dtype=jnp.int32)[:, None]
    pad_src = (i[None, :] * 41 + w * 977) % N
    pad_dst = N + (i[None, :] + w * (N_PAD - N) // NW) % (N_PAD - N)
    sp = jnp.concatenate([src.astype(jnp.int32).reshape(NW, per_w), pad_src], 1)
    dp = jnp.concatenate([dst.astype(jnp.int32).reshape(NW, per_w), pad_dst], 1)
    return sp.reshape(-1, K), dp.reshape(-1, K)


def kernel(x, adj, W1, b1, W2, b2):
    zrow = jnp.zeros((ROWS_PER_TILE, D), jnp.float32)
    zdeg = jnp.zeros((N_PAD,), jnp.float32)

    src1, dst1 = _pad_edges(adj[0, 0], adj[0, 1])
    src2, dst2 = _pad_edges(adj[1, 0], adj[1, 1])

    sc_aggregate = _sc_aggregate_kernel()
    h1 = _tc_pre(x, W1, b1)
    acc1, deg1 = sc_aggregate(h1, src1, dst1, zrow, zdeg)
    h2 = _tc_combine_pre(acc1.reshape(NC, N_PAD, D), deg1, h1, W2, b2)
    acc2, deg2 = sc_aggregate(h2, src2, dst2, zrow, zdeg)
    return _tc_combine(acc2.reshape(NC, N_PAD, D), deg2, h2)
